# Initial kernel scaffold; baseline (speedup 1.0000x reference)
import jax
import jax.numpy as jnp
from jax.experimental import pallas as pl


def kernel(feat, edge_index, W_neigh, W_self, bias):
    raise NotImplementedError("write your pallas kernel here")


# speedup vs baseline: 3.3742x; 3.3742x over previous
import functools

import jax
import jax.numpy as jnp
from jax import lax
from jax.experimental import pallas as pl
from jax.experimental.pallas import tpu as pltpu
from jax.experimental.pallas import tpu_sc as plsc

N_NODES = 10000
D = 128
NC, NS = 2, 16
NW = NC * NS
CHUNK = 128
N_ACC = 10240
ROWS_PER_TILE = N_ACC // NS
DEG_W = 16
SLAB = 8


def _sc_agg_body(src_hbm, dst_hbm, feat_hbm, sum_out, deg_out,
                 src_v, dst_v, rows_v, ones_v, zdeg_v, accum, degacc, sem):
    c = lax.axis_index("c")
    s = lax.axis_index("s")
    w = s * NC + c
    cpw = src_hbm.shape[0] // NW

    zero16 = jnp.zeros((16,), jnp.float32)
    one16 = jnp.ones((16,), jnp.float32)

    def z_rows(i, _):
        rows_v[i // 8, pl.ds((i % 8) * 16, 16)] = zero16
        return 0
    lax.fori_loop(0, CHUNK * 8, z_rows, 0)

    def z_deg(i, _):
        zdeg_v[i, pl.ds(0, 16)] = zero16
        return 0
    lax.fori_loop(0, CHUNK, z_deg, 0)

    def o_rows(i, _):
        ones_v[i, pl.ds(0, 16)] = one16
        return 0
    lax.fori_loop(0, CHUNK, o_rows, 0)

    base = s * ROWS_PER_TILE
    for k in range(ROWS_PER_TILE // CHUNK):
        pltpu.sync_copy(rows_v, accum.at[pl.ds(base + k * CHUNK, CHUNK)])
        pltpu.sync_copy(zdeg_v, degacc.at[pl.ds(base + k * CHUNK, CHUNK)])
    plsc.subcore_barrier()

    def outer(t, _):
        cbase = w * cpw + t * SLAB
        pltpu.sync_copy(src_hbm.at[pl.ds(cbase, SLAB)], src_v)
        pltpu.sync_copy(dst_hbm.at[pl.ds(cbase, SLAB)], dst_v)
        for j in range(SLAB):
            pltpu.async_copy(feat_hbm.at[src_v.at[j]], rows_v, sem).wait()
            pltpu.sync_copy(rows_v, accum.at[dst_v.at[j]], add=True)
            pltpu.sync_copy(ones_v, degacc.at[dst_v.at[j]], add=True)
        return 0
    lax.fori_loop(0, cpw // SLAB, outer, 0)

    plsc.subcore_barrier()

    pltpu.sync_copy(accum.at[pl.ds(base, ROWS_PER_TILE)],
                    sum_out.at[c, pl.ds(base, ROWS_PER_TILE)])
    pltpu.sync_copy(degacc.at[pl.ds(base, ROWS_PER_TILE)],
                    deg_out.at[c, pl.ds(base, ROWS_PER_TILE)])


def _make_sc_agg(n_chunks):
    cpw = n_chunks // NW
    return pl.kernel(
        _sc_agg_body,
        out_type=[
            jax.ShapeDtypeStruct((NC, N_ACC, D), jnp.float32),
            jax.ShapeDtypeStruct((NC, N_ACC, DEG_W), jnp.float32),
        ],
        mesh=plsc.VectorSubcoreMesh(core_axis_name="c", subcore_axis_name="s"),
        scratch_types=[
            pltpu.VMEM((SLAB, CHUNK), jnp.int32),
            pltpu.VMEM((SLAB, CHUNK), jnp.int32),
            pltpu.VMEM((CHUNK, D), jnp.float32),
            pltpu.VMEM((CHUNK, DEG_W), jnp.float32),
            pltpu.VMEM((CHUNK, DEG_W), jnp.float32),
            pltpu.VMEM_SHARED((N_ACC, D), jnp.float32),
            pltpu.VMEM_SHARED((N_ACC, DEG_W), jnp.float32),
            pltpu.SemaphoreType.DMA,
        ],
        compiler_params=pltpu.CompilerParams(use_tc_tiling_on_sc=False),
    )


def _tc_body(feat_ref, s_ref, d_ref, wn_ref, ws_ref, b_ref, out_ref):
    ssum = s_ref[0] + s_ref[1]
    deg = d_ref[0, :, 0:1] + d_ref[1, :, 0:1]
    h_neigh = ssum / jnp.maximum(deg, 1.0)
    out_ref[...] = (
        jnp.dot(h_neigh, wn_ref[...], preferred_element_type=jnp.float32)
        + jnp.dot(feat_ref[...], ws_ref[...], preferred_element_type=jnp.float32)
        + b_ref[...]
    )


def _tc_finish(feat_p, ssum, dacc, wn_t, ws_t, bias2d):
    blk = 1280
    grid = N_ACC // blk
    return pl.pallas_call(
        _tc_body,
        grid=(grid,),
        in_specs=[
            pl.BlockSpec((blk, D), lambda i: (i, 0)),
            pl.BlockSpec((NC, blk, D), lambda i: (0, i, 0)),
            pl.BlockSpec((NC, blk, DEG_W), lambda i: (0, i, 0)),
            pl.BlockSpec((D, D), lambda i: (0, 0)),
            pl.BlockSpec((D, D), lambda i: (0, 0)),
            pl.BlockSpec((1, D), lambda i: (0, 0)),
        ],
        out_specs=pl.BlockSpec((blk, D), lambda i: (i, 0)),
        out_shape=jax.ShapeDtypeStruct((N_ACC, D), jnp.float32),
    )(feat_p, ssum, dacc, wn_t, ws_t, bias2d)


@jax.jit
def kernel(feat, edge_index, W_neigh, W_self, bias):
    e = edge_index.shape[1]
    cpw = -(-(-(-e // (CHUNK * NW))) // 8) * 8
    n_chunks = cpw * NW
    e_pad = n_chunks * CHUNK
    src = edge_index[0].astype(jnp.int32)
    dst = edge_index[1].astype(jnp.int32)
    pad = e_pad - e
    src_p = jnp.concatenate([src, jnp.zeros((pad,), jnp.int32)]).reshape(n_chunks, CHUNK)
    dst_p = jnp.concatenate([dst, jnp.full((pad,), N_NODES, jnp.int32)]).reshape(n_chunks, CHUNK)

    ssum, dacc = _make_sc_agg(n_chunks)(src_p, dst_p, feat)

    feat_p = jnp.zeros((N_ACC, D), feat.dtype).at[:N_NODES].set(feat)
    out = _tc_finish(feat_p, ssum, dacc, W_neigh.T, W_self.T,
                     bias.reshape(1, D))
    return out[:N_NODES]


# speedup vs baseline: 3.7575x; 1.1136x over previous
import functools

import jax
import jax.numpy as jnp
from jax import lax
from jax.experimental import pallas as pl
from jax.experimental.pallas import tpu as pltpu
from jax.experimental.pallas import tpu_sc as plsc

N_NODES = 10000
D = 128
NC, NS = 2, 16
NW = NC * NS
CHUNK = 128
N_ACC = 10112
ROWS_PER_TILE = N_ACC // NS
DEG_W = 16
SLAB = 8


def _sc_agg_body(src_hbm, dst_hbm, feat_hbm, sum_out, deg_out,
                 src_v, dst_v, rows_a, rows_b, ones_v, zdeg_v,
                 accum, degacc, gsem_a, gsem_b, ssem_a, ssem_b):
    c = lax.axis_index("c")
    s = lax.axis_index("s")
    w = s * NC + c
    cpw = src_hbm.shape[0] // NW

    zero16 = jnp.zeros((16,), jnp.float32)
    one16 = jnp.ones((16,), jnp.float32)

    def z_rows(i, _):
        rows_a[i // 8, pl.ds((i % 8) * 16, 16)] = zero16
        return 0
    lax.fori_loop(0, CHUNK * 8, z_rows, 0)

    def z_deg(i, _):
        zdeg_v[i, pl.ds(0, 16)] = zero16
        return 0
    lax.fori_loop(0, CHUNK, z_deg, 0)

    def o_rows(i, _):
        ones_v[i, pl.ds(0, 16)] = one16
        return 0
    lax.fori_loop(0, CHUNK, o_rows, 0)

    base = s * ROWS_PER_TILE
    for k in range(ROWS_PER_TILE // CHUNK):
        pltpu.sync_copy(rows_a, accum.at[pl.ds(base + k * CHUNK, CHUNK)])
        pltpu.sync_copy(zdeg_v, degacc.at[pl.ds(base + k * CHUNK, CHUNK)])
    rem = ROWS_PER_TILE % CHUNK
    if rem:
        rbase = base + (ROWS_PER_TILE // CHUNK) * CHUNK
        pltpu.sync_copy(rows_a.at[pl.ds(0, rem)], accum.at[pl.ds(rbase, rem)])
        pltpu.sync_copy(zdeg_v.at[pl.ds(0, rem)], degacc.at[pl.ds(rbase, rem)])
    plsc.subcore_barrier()

    bufs = (rows_a, rows_b)
    gsems = (gsem_a, gsem_b)
    ssems = (ssem_a, ssem_b)

    def outer(t, _):
        cbase = w * cpw + t * SLAB
        pltpu.sync_copy(src_hbm.at[pl.ds(cbase, SLAB)], src_v)
        pltpu.sync_copy(dst_hbm.at[pl.ds(cbase, SLAB)], dst_v)
        gather = {0: pltpu.async_copy(feat_hbm.at[src_v.at[0]], bufs[0], gsems[0])}
        pend = {0: [], 1: []}
        for j in range(SLAB):
            b = j % 2
            nb = 1 - b
            if j + 1 < SLAB:
                for h in pend[nb]:
                    h.wait()
                pend[nb] = []
                gather[nb] = pltpu.async_copy(
                    feat_hbm.at[src_v.at[j + 1]], bufs[nb], gsems[nb])
            gather[b].wait()
            pend[b] = [
                pltpu.async_copy(bufs[b], accum.at[dst_v.at[j]], ssems[b], add=True),
                pltpu.async_copy(ones_v, degacc.at[dst_v.at[j]], ssems[b], add=True),
            ]
        for k in (0, 1):
            for h in pend[k]:
                h.wait()
        return 0
    lax.fori_loop(0, cpw // SLAB, outer, 0)

    plsc.subcore_barrier()

    pltpu.sync_copy(accum.at[pl.ds(base, ROWS_PER_TILE)],
                    sum_out.at[c, pl.ds(base, ROWS_PER_TILE)])
    pltpu.sync_copy(degacc.at[pl.ds(base, ROWS_PER_TILE)],
                    deg_out.at[c, pl.ds(base, ROWS_PER_TILE)])


def _make_sc_agg(n_chunks):
    return pl.kernel(
        _sc_agg_body,
        out_type=[
            jax.ShapeDtypeStruct((NC, N_ACC, D), jnp.float32),
            jax.ShapeDtypeStruct((NC, N_ACC, DEG_W), jnp.float32),
        ],
        mesh=plsc.VectorSubcoreMesh(core_axis_name="c", subcore_axis_name="s"),
        scratch_types=[
            pltpu.VMEM((SLAB, CHUNK), jnp.int32),
            pltpu.VMEM((SLAB, CHUNK), jnp.int32),
            pltpu.VMEM((CHUNK, D), jnp.float32),
            pltpu.VMEM((CHUNK, D), jnp.float32),
            pltpu.VMEM((CHUNK, DEG_W), jnp.float32),
            pltpu.VMEM((CHUNK, DEG_W), jnp.float32),
            pltpu.VMEM_SHARED((N_ACC, D), jnp.float32),
            pltpu.VMEM_SHARED((N_ACC, DEG_W), jnp.float32),
            pltpu.SemaphoreType.DMA,
            pltpu.SemaphoreType.DMA,
            pltpu.SemaphoreType.DMA,
            pltpu.SemaphoreType.DMA,
        ],
        compiler_params=pltpu.CompilerParams(use_tc_tiling_on_sc=False),
    )


def _tc_body(feat_ref, s_ref, d_ref, wn_ref, ws_ref, b_ref, out_ref):
    ssum = s_ref[0] + s_ref[1]
    deg = d_ref[0, :, 0:1] + d_ref[1, :, 0:1]
    h_neigh = ssum / jnp.maximum(deg, 1.0)
    out_ref[...] = (
        jnp.dot(h_neigh, wn_ref[...], preferred_element_type=jnp.float32)
        + jnp.dot(feat_ref[...], ws_ref[...], preferred_element_type=jnp.float32)
        + b_ref[...]
    )


def _tc_finish(feat_p, ssum, dacc, wn_t, ws_t, bias2d):
    blk = 1264
    grid = N_ACC // blk
    return pl.pallas_call(
        _tc_body,
        grid=(grid,),
        in_specs=[
            pl.BlockSpec((blk, D), lambda i: (i, 0)),
            pl.BlockSpec((NC, blk, D), lambda i: (0, i, 0)),
            pl.BlockSpec((NC, blk, DEG_W), lambda i: (0, i, 0)),
            pl.BlockSpec((D, D), lambda i: (0, 0)),
            pl.BlockSpec((D, D), lambda i: (0, 0)),
            pl.BlockSpec((1, D), lambda i: (0, 0)),
        ],
        out_specs=pl.BlockSpec((blk, D), lambda i: (i, 0)),
        out_shape=jax.ShapeDtypeStruct((N_ACC, D), jnp.float32),
    )(feat_p, ssum, dacc, wn_t, ws_t, bias2d)


@jax.jit
def kernel(feat, edge_index, W_neigh, W_self, bias):
    e = edge_index.shape[1]
    cpw = -(-(-(-e // (CHUNK * NW))) // SLAB) * SLAB
    n_chunks = cpw * NW
    e_pad = n_chunks * CHUNK
    src = edge_index[0].astype(jnp.int32)
    dst = edge_index[1].astype(jnp.int32)
    pad = e_pad - e
    src_p = jnp.concatenate([src, jnp.zeros((pad,), jnp.int32)]).reshape(n_chunks, CHUNK)
    dst_p = jnp.concatenate([dst, jnp.full((pad,), N_NODES, jnp.int32)]).reshape(n_chunks, CHUNK)

    ssum, dacc = _make_sc_agg(n_chunks)(src_p, dst_p, feat)

    feat_p = jnp.zeros((N_ACC, D), feat.dtype).at[:N_NODES].set(feat)
    out = _tc_finish(feat_p, ssum, dacc, W_neigh.T, W_self.T,
                     bias.reshape(1, D))
    return out[:N_NODES]
